# Initial kernel scaffold; baseline (speedup 1.0000x reference)
import jax
import jax.numpy as jnp
from jax.experimental import pallas as pl


def kernel(mol_x, mol_edge_index, mol_batch, pro_x, pro_edge_index, pro_orig_edge_index, pro_batch, mc1_W, mc1_b, mc2_W, mc2_b, mc3_W, mc3_b, mfc1_W, mfc1_b, mfc2_W, mfc2_b, pc1_W, pc1_b, pc2_W, pc2_b, pc3_W, pc3_b, pfc1_W, pfc1_b, pfc2_W, pfc2_b, fc1_W, fc1_b, fc2_W, fc2_b, out_W, out_b):
    raise NotImplementedError("write your pallas kernel here")


# speedup vs baseline: 2.7717x; 2.7717x over previous
import functools

import jax
import jax.numpy as jnp
from jax import lax
from jax.experimental import pallas as pl
from jax.experimental.pallas import tpu as pltpu
from jax.experimental.pallas import tpu_sc as plsc

N = 50000
E = 800000
NG = 128
NPAD = 51200
EP = 819200
EPW = EP // 32
EB = 1024
GB = 128
W = 128
NQ = 8
QN = NPAD // NQ
CAPP = 25728
R = 512
STRIPE = QN // 16


def _sc_mesh():
    return plsc.VectorSubcoreMesh(
        core_axis_name="c", subcore_axis_name="s", num_cores=2, num_subcores=16)


@functools.cache
def _make_scatter_sc(nc, with_gather):
    scratch = [
        pltpu.VMEM_SHARED((QN + 8, W), jnp.float32),
        pltpu.VMEM((EB,), jnp.int32),
        pltpu.VMEM((8, 128), jnp.int32),
        pltpu.VMEM((CAPP,), jnp.int32),
        pltpu.VMEM((CAPP,), jnp.int32),
        pltpu.VMEM((GB, W), jnp.float32),
        pltpu.VMEM((16, W), jnp.float32),
        pltpu.SemaphoreType.DMA,
    ]

    @functools.partial(
        pl.kernel,
        out_type=jax.ShapeDtypeStruct((2 * nc * NPAD, W), jnp.float32),
        mesh=_sc_mesh(),
        scratch_types=scratch,
        compiler_params=pltpu.CompilerParams(needs_layout_passes=False),
    )
    def conv(g_hbm, src_hbm, dst2d_hbm, out_hbm, slab, sidx, didx, csrc,
             cdst, rows, zbuf, sem):
        cid = lax.axis_index("c")
        sid = lax.axis_index("s")
        wid = sid * 2 + cid
        zvec = jnp.zeros((16,), jnp.float32)
        for r in range(16):
            for t in range(W // 16):
                zbuf[r, pl.ds(t * 16, 16)] = zvec
        if not with_gather:
            pltpu.sync_copy(g_hbm, rows)
        stripe0 = sid * STRIPE
        ebase = wid * EPW
        dummy_d = jnp.full((16,), QN, jnp.int32)
        dummy_s = jnp.zeros((16,), jnp.int32)

        @pl.loop(0, NQ)
        def _quarter(q):
            qlo = q * QN

            @pl.loop(0, EPW // EB, init_carry=jnp.int32(0))
            def _blkA(b, off):
                e0 = pl.multiple_of(ebase + b * EB, 8)
                pltpu.sync_copy(src_hbm.at[pl.ds(e0, EB)], sidx)
                r0 = pl.multiple_of(e0 // 128, 8)
                pltpu.sync_copy(dst2d_hbm.at[pl.ds(r0, 8)], didx)
                for j in range(8):
                    for v in range(8):
                        sv = sidx[pl.ds(j * 128 + v * 16, 16)]
                        dv = didx[j, pl.ds(v * 16, 16)]
                        dl = dv - qlo
                        m = (dl >= 0) & (dl < QN)
                        inc = m.astype(jnp.int32)
                        pos = off + plsc.cumsum(inc) - 1
                        if with_gather:
                            plsc.store_scatter(csrc, [pos], sv, mask=m)
                        plsc.store_scatter(cdst, [pos], dl, mask=m)
                        off = off + jnp.sum(inc)
                return off

            off = _blkA
            for t in range(GB // 16):
                if with_gather:
                    csrc[pl.ds(off + t * 16, 16)] = dummy_s
                cdst[pl.ds(off + t * 16, 16)] = dummy_d
            nblk = lax.div(off + GB - 1, jnp.int32(GB))

            for c in range(nc):
                if with_gather and c > 0:
                    @pl.loop(0, CAPP // 16)
                    def _bump(i):
                        p = i * 16
                        csrc[pl.ds(p, 16)] = csrc[pl.ds(p, 16)] + NPAD
                for k in range(STRIPE // 16):
                    pltpu.sync_copy(zbuf, slab.at[pl.ds(stripe0 + k * 16, 16)])
                plsc.subcore_barrier()

                @pl.loop(0, nblk)
                def _blkB(b):
                    bb = b * GB
                    if with_gather:
                        pltpu.async_copy(g_hbm.at[csrc.at[pl.ds(bb, GB)]],
                                         rows, sem).wait()
                    for j in range(GB // 128):
                        pltpu.sync_copy(
                            rows.at[pl.ds(j * 128, 128)],
                            slab.at[cdst.at[pl.ds(bb + j * 128, 128)]],
                            add=True)

                plsc.subcore_barrier()
                row0 = pl.multiple_of(
                    (cid * nc + c) * NPAD + qlo + stripe0, 8)
                pltpu.sync_copy(slab.at[pl.ds(stripe0, STRIPE)],
                                out_hbm.at[pl.ds(row0, STRIPE)])
                plsc.subcore_barrier()

    return conv


def _dot(a, b):
    return jnp.dot(a, b, preferred_element_type=jnp.float32)


@functools.cache
def _make_dinv_tc():
    def body(deg_ref, o_ref):
        d = deg_ref[0, :, 0:1] + deg_ref[1, :, 0:1] + 1.0
        o_ref[...] = lax.rsqrt(jnp.maximum(d, 1.0))

    return pl.pallas_call(
        body,
        grid=(NPAD // R,),
        in_specs=[pl.BlockSpec((2, R, W), lambda i: (0, i, 0))],
        out_specs=pl.BlockSpec((R, 1), lambda i: (i, 0)),
        out_shape=jax.ShapeDtypeStruct((NPAD, 1), jnp.float32),
    )


@functools.cache
def _make_mm_pre(kp, nc):
    def body(x_ref, w_ref, dinv_ref, o_ref):
        h = _dot(x_ref[...], w_ref[...])
        dv = dinv_ref[...]
        for c in range(nc):
            o_ref[c] = dv * h[:, c * W:(c + 1) * W]

    return pl.pallas_call(
        body,
        grid=(NPAD // R,),
        in_specs=[
            pl.BlockSpec((R, kp), lambda i: (i, 0)),
            pl.BlockSpec((kp, nc * W), lambda i: (0, 0)),
            pl.BlockSpec((R, 1), lambda i: (i, 0)),
        ],
        out_specs=pl.BlockSpec((nc, R, W), lambda i: (0, i, 0)),
        out_shape=jax.ShapeDtypeStruct((nc, NPAD, W), jnp.float32),
    )


@functools.cache
def _make_fuse(nc, ncn):
    def body(s_ref, g_ref, dinv_ref, b_ref, w_ref, dinv2_ref, o_ref):
        dv = dinv_ref[...]
        acc = None
        for c in range(nc):
            xc = s_ref[0, c] + s_ref[1, c] + g_ref[c]
            xc = jnp.maximum(dv * xc + b_ref[c], 0.0)
            p = _dot(xc, w_ref[c])
            acc = p if acc is None else acc + p
        dv2 = dinv2_ref[...]
        for c in range(ncn):
            o_ref[c] = dv2 * acc[:, c * W:(c + 1) * W]

    return pl.pallas_call(
        body,
        grid=(NPAD // R,),
        in_specs=[
            pl.BlockSpec((2, nc, R, W), lambda i: (0, 0, i, 0)),
            pl.BlockSpec((nc, R, W), lambda i: (0, i, 0)),
            pl.BlockSpec((R, 1), lambda i: (i, 0)),
            pl.BlockSpec((nc, 1, W), lambda i: (0, 0, 0)),
            pl.BlockSpec((nc, W, ncn * W), lambda i: (0, 0, 0)),
            pl.BlockSpec((R, 1), lambda i: (i, 0)),
        ],
        out_specs=pl.BlockSpec((ncn, R, W), lambda i: (0, i, 0)),
        out_shape=jax.ShapeDtypeStruct((ncn, NPAD, W), jnp.float32),
    )


@functools.cache
def _make_pool(nc):
    def body(s_ref, g_ref, dinv_ref, b_ref, batch_ref, ps_ref, cnt_ref):
        i = pl.program_id(0)

        @pl.when(i == 0)
        def _():
            ps_ref[...] = jnp.zeros_like(ps_ref)
            cnt_ref[...] = jnp.zeros_like(cnt_ref)

        gid = jax.lax.broadcasted_iota(jnp.int32, (1, NG), 1)
        oh = (batch_ref[...] == gid).astype(jnp.float32)
        ones = jnp.ones((R, W), jnp.float32)
        cnt_ref[...] += jax.lax.dot_general(
            oh, ones, (((0,), (0,)), ((), ())),
            preferred_element_type=jnp.float32)
        dv = dinv_ref[...]
        for c in range(nc):
            xc = s_ref[0, c] + s_ref[1, c] + g_ref[c]
            xc = jnp.maximum(dv * xc + b_ref[c], 0.0)
            ps_ref[c] += jax.lax.dot_general(
                oh, xc, (((0,), (0,)), ((), ())),
                preferred_element_type=jnp.float32)

    return pl.pallas_call(
        body,
        grid=(NPAD // R,),
        in_specs=[
            pl.BlockSpec((2, nc, R, W), lambda i: (0, 0, i, 0)),
            pl.BlockSpec((nc, R, W), lambda i: (0, i, 0)),
            pl.BlockSpec((R, 1), lambda i: (i, 0)),
            pl.BlockSpec((nc, 1, W), lambda i: (0, 0, 0)),
            pl.BlockSpec((R, 1), lambda i: (i, 0)),
        ],
        out_specs=[
            pl.BlockSpec((nc, NG, W), lambda i: (0, 0, 0)),
            pl.BlockSpec((NG, W), lambda i: (0, 0)),
        ],
        out_shape=[
            jax.ShapeDtypeStruct((nc, NG, W), jnp.float32),
            jax.ShapeDtypeStruct((NG, W), jnp.float32),
        ],
    )


@functools.cache
def _make_head(ncm, ncp):
    def body(mps_ref, mcnt_ref, pps_ref, pcnt_ref,
             mf1w_ref, mf1b_ref, mf2w_ref, mf2b_ref,
             pf1w_ref, pf1b_ref, pf2w_ref, pf2b_ref,
             f1w_ref, f1b_ref, f2w_ref, f2b_ref, ow_ref, ob_ref, o_ref):
        mr = 1.0 / jnp.maximum(mcnt_ref[...], 1.0)
        xm = None
        for c in range(ncm):
            p = _dot(mps_ref[c] * mr, mf1w_ref[c])
            xm = p if xm is None else xm + p
        xm = jnp.maximum(xm + mf1b_ref[...], 0.0)
        xm = _dot(xm, mf2w_ref[...]) + mf2b_ref[...]
        pr = 1.0 / jnp.maximum(pcnt_ref[...], 1.0)
        xp = None
        for c in range(ncp):
            p = _dot(pps_ref[c] * pr, pf1w_ref[c])
            xp = p if xp is None else xp + p
        xp = jnp.maximum(xp + pf1b_ref[...], 0.0)
        xp = _dot(xp, pf2w_ref[...]) + pf2b_ref[...]
        xc = jnp.concatenate([xp, xm], axis=1)
        h = jnp.maximum(_dot(xc, f1w_ref[...]) + f1b_ref[...], 0.0)
        h = jnp.maximum(_dot(h, f2w_ref[...]) + f2b_ref[...], 0.0)
        o_ref[...] = _dot(h, ow_ref[...]) + ob_ref[...]

    return pl.pallas_call(
        body,
        out_shape=jax.ShapeDtypeStruct((NG, 128), jnp.float32),
    )


def _pad2(a, rows, cols):
    return jnp.pad(a, ((0, rows - a.shape[0]), (0, cols - a.shape[1])))


def _edges(ei):
    pad = jnp.full((EP - E,), N, jnp.int32)
    src = jnp.concatenate([ei[0], pad])
    dst = jnp.concatenate([ei[1], pad])
    return src, dst.reshape(EP // 128, 128)


def _wchunk(w, kp, fp):
    return _pad2(w, kp, fp).reshape(kp // W, W, fp)


def _bchunk(b, fp):
    return jnp.pad(b, (0, fp - b.shape[0])).reshape(fp // W, 1, W)


def _deg_dinv(src, dst2d, ones_rows):
    deg = _make_scatter_sc(1, False)(ones_rows, src, dst2d)
    return _make_dinv_tc()(deg.reshape(2, NPAD, W))


def _branch(x, src1, dst2d1, src23, dst2d23, dinv1, dinv23, batch,
            w1, b1, w2, b2, w3, b3):
    f1p = -(-w1.shape[1] // W) * W
    f2p = -(-w2.shape[1] // W) * W
    f3p = -(-w3.shape[1] // W) * W
    nc1, nc2, nc3 = f1p // W, f2p // W, f3p // W

    xp = _pad2(x, NPAD, W)
    g1 = _make_mm_pre(W, nc1)(xp, _pad2(w1, W, f1p), dinv1)
    s1 = _make_scatter_sc(nc1, True)(g1.reshape(nc1 * NPAD, W), src1,
                                     dst2d1).reshape(2, nc1, NPAD, W)
    g2 = _make_fuse(nc1, nc2)(s1, g1, dinv1, _bchunk(b1, f1p),
                              _wchunk(w2, f1p, f2p), dinv23)
    s2 = _make_scatter_sc(nc2, True)(g2.reshape(nc2 * NPAD, W), src23,
                                     dst2d23).reshape(2, nc2, NPAD, W)
    g3 = _make_fuse(nc2, nc3)(s2, g2, dinv23, _bchunk(b2, f2p),
                              _wchunk(w3, f2p, f3p), dinv23)
    s3 = _make_scatter_sc(nc3, True)(g3.reshape(nc3 * NPAD, W), src23,
                                     dst2d23).reshape(2, nc3, NPAD, W)
    bp = jnp.concatenate(
        [batch, jnp.full((NPAD - N,), NG, jnp.int32)]).reshape(NPAD, 1)
    ps, cnt = _make_pool(nc3)(s3, g3, dinv23, _bchunk(b3, f3p), bp)
    return ps, cnt, nc3


def kernel(mol_x, mol_edge_index, mol_batch, pro_x, pro_edge_index,
           pro_orig_edge_index, pro_batch, mc1_W, mc1_b, mc2_W, mc2_b, mc3_W,
           mc3_b, mfc1_W, mfc1_b, mfc2_W, mfc2_b, pc1_W, pc1_b, pc2_W, pc2_b,
           pc3_W, pc3_b, pfc1_W, pfc1_b, pfc2_W, pfc2_b, fc1_W, fc1_b, fc2_W,
           fc2_b, out_W, out_b):
    src_m, dst2d_m = _edges(mol_edge_index)
    src_po, dst2d_po = _edges(pro_orig_edge_index)
    src_p, dst2d_p = _edges(pro_edge_index)

    ones_rows = jnp.ones((GB, W), jnp.float32)
    dinv_m = _deg_dinv(src_m, dst2d_m, ones_rows)
    dinv_po = _deg_dinv(src_po, dst2d_po, ones_rows)
    dinv_p = _deg_dinv(src_p, dst2d_p, ones_rows)

    mps, mcnt, ncm3 = _branch(mol_x, src_m, dst2d_m, src_m, dst2d_m,
                              dinv_m, dinv_m, mol_batch,
                              mc1_W, mc1_b, mc2_W, mc2_b, mc3_W, mc3_b)
    pps, pcnt, ncp3 = _branch(pro_x, src_po, dst2d_po, src_p, dst2d_p,
                              dinv_po, dinv_p, pro_batch,
                              pc1_W, pc1_b, pc2_W, pc2_b, pc3_W, pc3_b)
    out = _make_head(ncm3, ncp3)(
        mps, mcnt, pps, pcnt,
        _wchunk(mfc1_W, ncm3 * W, 1024), mfc1_b.reshape(1, 1024),
        mfc2_W, mfc2_b.reshape(1, 128),
        _wchunk(pfc1_W, ncp3 * W, 1024), pfc1_b.reshape(1, 1024),
        pfc2_W, pfc2_b.reshape(1, 128),
        fc1_W, fc1_b.reshape(1, 1024),
        fc2_W, fc2_b.reshape(1, 512),
        _pad2(out_W, 512, 128), jnp.pad(out_b, (0, 127)).reshape(1, 128))
    return out[:, :1]
